# Initial kernel scaffold; baseline (speedup 1.0000x reference)
import jax
import jax.numpy as jnp
from jax.experimental import pallas as pl


def kernel(x, edge_index, W1, b1, W2, b2, W3, b3, Wc1, bc1, Wc2, bc2):
    raise NotImplementedError("write your pallas kernel here")


# speedup vs baseline: 11.3782x; 11.3782x over previous
import functools

import jax
import jax.numpy as jnp
from jax import lax
from jax.experimental import pallas as pl
from jax.experimental.pallas import tpu as pltpu
from jax.experimental.pallas import tpu_sc as plsc

N = 10000
E = 320000
D = 128
NSC = 2
NTILE = 16
EB = 128
NB = 79
E_PAD = NSC * NTILE * NB * EB
CHUNK = 624
TAIL0 = CHUNK * NTILE
TAIL = N - TAIL0
N_DEG = 10240
CHUNK_DEG = N_DEG // NTILE
ACC_ROWS = N + 16
TRASH = N

_mesh = plsc.VectorSubcoreMesh(core_axis_name="c", subcore_axis_name="s")


def _sc_deg_body(dstp_hbm, ones_hbm, zeros_hbm, out_hbm, dst_v, ones_v, acc):
    c = lax.axis_index("c")
    s = lax.axis_index("s")
    w = c * NTILE + s
    r0 = s * CHUNK_DEG
    pltpu.sync_copy(dstp_hbm.at[w], dst_v)
    pltpu.sync_copy(ones_hbm, ones_v)
    pltpu.sync_copy(zeros_hbm.at[pl.ds(r0, CHUNK_DEG)],
                    acc.at[pl.ds(r0, CHUNK_DEG)])
    plsc.subcore_barrier()

    @pl.loop(0, NB)
    def _(j):
        pltpu.sync_copy(ones_v, acc.at[dst_v.at[j]], add=True)

    plsc.subcore_barrier()
    pltpu.sync_copy(acc.at[pl.ds(r0, CHUNK_DEG)],
                    out_hbm.at[c, 0, pl.ds(r0, CHUNK_DEG)])


def _sc_deg(dstp, ones, zeros):
    return pl.kernel(
        _sc_deg_body,
        out_type=jax.ShapeDtypeStruct((NSC, 1, N_DEG), jnp.float32),
        mesh=_mesh,
        scratch_types=[
            pltpu.VMEM((NB, EB), jnp.int32),
            pltpu.VMEM((EB,), jnp.float32),
            pltpu.VMEM_SHARED((N_DEG + 128,), jnp.float32),
        ],
    )(dstp, ones, zeros)


def _sc_agg_body(ht_hbm, srcp_hbm, dstp_hbm, out_hbm, src_v, dst_v, buf, acc):
    c = lax.axis_index("c")
    s = lax.axis_index("s")
    w = c * NTILE + s
    r0 = s * CHUNK
    pltpu.sync_copy(srcp_hbm.at[w], src_v)
    pltpu.sync_copy(dstp_hbm.at[w], dst_v)
    pltpu.sync_copy(ht_hbm.at[pl.ds(r0, CHUNK)], acc.at[pl.ds(r0, CHUNK)])

    @pl.when(s == NTILE - 1)
    def _():
        pltpu.sync_copy(ht_hbm.at[pl.ds(TAIL0, TAIL)],
                        acc.at[pl.ds(TAIL0, TAIL)])

    plsc.subcore_barrier()

    @pl.loop(0, NB)
    def _(j):
        pltpu.sync_copy(ht_hbm.at[src_v.at[j]], buf)
        pltpu.sync_copy(buf, acc.at[dst_v.at[j]], add=True)

    plsc.subcore_barrier()
    pltpu.sync_copy(acc.at[pl.ds(r0, CHUNK)],
                    out_hbm.at[c, pl.ds(r0, CHUNK), :])

    @pl.when(s == NTILE - 1)
    def _():
        pltpu.sync_copy(acc.at[pl.ds(TAIL0, TAIL)],
                        out_hbm.at[c, pl.ds(TAIL0, TAIL), :])


def _sc_agg(ht, srcp, dstp):
    return pl.kernel(
        _sc_agg_body,
        out_type=jax.ShapeDtypeStruct((NSC, N, D), jnp.float32),
        mesh=_mesh,
        scratch_types=[
            pltpu.VMEM((NB, EB), jnp.int32),
            pltpu.VMEM((NB, EB), jnp.int32),
            pltpu.VMEM((EB, D), jnp.float32),
            pltpu.VMEM_SHARED((ACC_ROWS, D), jnp.float32),
        ],
    )(ht, srcp, dstp)


_DOT = dict(precision=lax.Precision.HIGHEST, preferred_element_type=jnp.float32)


def _tc_first_body(x_ref, w_ref, deg_ref, ht_ref, dinv_ref):
    deg = deg_ref[0] + deg_ref[1] + 1.0
    dinv = lax.rsqrt(deg)
    dinv_ref[...] = dinv
    ht_ref[...] = jnp.dot(x_ref[...], w_ref[...], **_DOT) * dinv


def _tc_first(x, W1, deg2):
    return pl.pallas_call(
        _tc_first_body,
        out_shape=(jax.ShapeDtypeStruct((N, D), jnp.float32),
                   jax.ShapeDtypeStruct((N, 1), jnp.float32)),
    )(x, W1, deg2)


def _tc_mid_body(agg_ref, ht_ref, dinv_ref, b_ref, w_ref, out_ref):
    a = agg_ref[0] + agg_ref[1] - ht_ref[...]
    dinv = dinv_ref[...]
    t = jnp.maximum(a * dinv + b_ref[...], 0.0)
    out_ref[...] = jnp.dot(t, w_ref[...], **_DOT) * dinv


def _tc_mid(agg, ht_prev, dinv, b, W):
    return pl.pallas_call(
        _tc_mid_body,
        out_shape=jax.ShapeDtypeStruct((N, D), jnp.float32),
    )(agg, ht_prev, dinv, b, W)


def _tc_head_body(agg_ref, ht_ref, dinv_ref, b3_ref, wc1_ref, bc1_ref,
                  wc2_ref, bc2_ref, out_ref):
    a = agg_ref[0] + agg_ref[1] - ht_ref[...]
    h3 = jnp.maximum(a * dinv_ref[...] + b3_ref[...], 0.0)
    pooled = jnp.sum(h3, axis=0, keepdims=True) * (1.0 / N)
    hidden = jnp.maximum(jnp.dot(pooled, wc1_ref[...], **_DOT) + bc1_ref[...],
                         0.0)
    out_ref[...] = jnp.dot(hidden, wc2_ref[...], **_DOT) + bc2_ref[...]


def _tc_head(agg, ht_prev, dinv, b3, Wc1, bc1, Wc2, bc2):
    return pl.pallas_call(
        _tc_head_body,
        out_shape=jax.ShapeDtypeStruct((1, 2), jnp.float32),
    )(agg, ht_prev, dinv, b3, Wc1, bc1, Wc2, bc2)


@jax.jit
def kernel(x, edge_index, W1, b1, W2, b2, W3, b3, Wc1, bc1, Wc2, bc2):
    src = edge_index[0].astype(jnp.int32)
    dst = edge_index[1].astype(jnp.int32)
    npad = E_PAD - E
    srcp = jnp.concatenate([src, jnp.zeros((npad,), jnp.int32)])
    dstp = jnp.concatenate([dst, jnp.full((npad,), TRASH, jnp.int32)])
    srcp = srcp.reshape(NSC * NTILE, NB, EB)
    dstp = dstp.reshape(NSC * NTILE, NB, EB)
    ones = jnp.ones((EB,), jnp.float32)
    zeros = jnp.zeros((N_DEG,), jnp.float32)

    deg2 = _sc_deg(dstp, ones, zeros)[:, 0, :N].reshape(NSC, N, 1)
    ht1, dinv = _tc_first(x, W1, deg2)
    agg1 = _sc_agg(ht1, srcp, dstp)
    ht2 = _tc_mid(agg1, ht1, dinv, b1.reshape(1, D), W2)
    agg2 = _sc_agg(ht2, srcp, dstp)
    ht3 = _tc_mid(agg2, ht2, dinv, b2.reshape(1, D), W3)
    agg3 = _sc_agg(ht3, srcp, dstp)
    return _tc_head(agg3, ht3, dinv, b3.reshape(1, D), Wc1,
                    bc1.reshape(1, D), Wc2, bc2.reshape(1, 2))
